# Initial kernel scaffold; baseline (speedup 1.0000x reference)
import jax
import jax.numpy as jnp
from jax.experimental import pallas as pl


def kernel(v, time_step, batch, u, log_alphas_bar, log_1_min_alphas_bar):
    raise NotImplementedError("write your pallas kernel here")


# speedup vs baseline: 4.7530x; 4.7530x over previous
import numpy as np
import jax
import jax.numpy as jnp
from jax.experimental import pallas as pl
from jax.experimental.pallas import tpu as pltpu

_NCLS = 64
_T = 100
_TPAD = 128
_LOG_NC = float(np.log(_NCLS))


def _block_body(ts_ref, la_ref, l1ma_ref, v_ref, b_ref, u_ref,
                vp_ref, lnvt_ref, lv0_ref):
    f32 = jnp.float32
    log_eps = jnp.log(f32(1e-30))

    def lae(a, b):
        m = jnp.maximum(a, b)
        return m + jnp.log(jnp.exp(a - m) + jnp.exp(b - m))

    la = la_ref[...]
    l1ma = l1ma_ref[...]
    rest = l1ma - _LOG_NC
    on_row = lae(la, rest)
    off_row = lae(la + log_eps, rest)

    ts = ts_ref[...]
    bidx = b_ref[...]
    vcls = v_ref[...]
    iota64 = jax.lax.broadcasted_iota(jnp.int32, (1, _NCLS), 1)
    iota128 = jax.lax.broadcasted_iota(jnp.int32, (1, _TPAD), 1)

    t_n = jnp.sum(jnp.where(bidx == iota64, ts, 0), axis=1, keepdims=True)
    mt = t_n == iota128
    on_n = jnp.sum(jnp.where(mt, on_row, f32(0.0)), axis=1, keepdims=True)
    off_n = jnp.sum(jnp.where(mt, off_row, f32(0.0)), axis=1, keepdims=True)

    u = u_ref[...]
    g = -jnp.log(-jnp.log(u + f32(1e-30)) + f32(1e-30))
    mv = vcls == iota64
    val = g + jnp.where(mv, on_n, off_n)
    vmax = jnp.max(val, axis=1, keepdims=True)
    samp = jnp.min(jnp.where(val == vmax, iota64, _NCLS), axis=1, keepdims=True)
    ms = samp == iota64

    vp_ref[...] = jnp.where(ms, f32(1.0), f32(0.0))
    lnvt_ref[...] = jnp.where(ms, f32(0.0), log_eps)
    lv0_ref[...] = jnp.where(mv, f32(0.0), log_eps)


def kernel(v, time_step, batch, u, log_alphas_bar, log_1_min_alphas_bar):
    n = u.shape[0]
    rows = 1024
    grid = n // rows
    ts2 = time_step.reshape(1, _NCLS)
    la2 = jnp.pad(log_alphas_bar, (0, _TPAD - _T)).reshape(1, _TPAD)
    l12 = jnp.pad(log_1_min_alphas_bar, (0, _TPAD - _T)).reshape(1, _TPAD)
    v2 = v.reshape(n, 1)
    b2 = batch.reshape(n, 1)

    grid_spec = pl.GridSpec(
        grid=(grid,),
        in_specs=[
            pl.BlockSpec((1, _NCLS), lambda i: (0, 0)),
            pl.BlockSpec((1, _TPAD), lambda i: (0, 0)),
            pl.BlockSpec((1, _TPAD), lambda i: (0, 0)),
            pl.BlockSpec((rows, 1), lambda i: (i, 0)),
            pl.BlockSpec((rows, 1), lambda i: (i, 0)),
            pl.BlockSpec((rows, _NCLS), lambda i: (i, 0)),
        ],
        out_specs=[pl.BlockSpec((rows, _NCLS), lambda i: (i, 0))] * 3,
    )
    vp, lnvt, lv0 = pl.pallas_call(
        _block_body,
        grid_spec=grid_spec,
        out_shape=[jax.ShapeDtypeStruct((n, _NCLS), jnp.float32)] * 3,
        compiler_params=pltpu.CompilerParams(
            dimension_semantics=("parallel",)),
    )(ts2, la2, l12, v2, b2, u)
    return (vp, lnvt, lv0)
